# baseline (speedup 1.0000x reference)
import numpy as np
import jax
import jax.numpy as jnp
from jax import lax
from jax.experimental import pallas as pl
from jax.experimental.pallas import tpu as pltpu, tpu_sc as plsc

F = 9
V = 124
D = 128
N = 50000
NPAIR = 4
VP = V * V
NC, NS = 2, 16
NW = NC * NS
NB = 16
BLK = 98
NPW = NB * BLK
TAIL = N - (NW - 1) * NPW
NBUF = 4
NG = NPAIR + 1

_PERM = np.empty((D,), dtype=np.int32)
for _g in range(D // 32):
    for _j in range(16):
        _PERM[_g * 32 + 2 * _j] = _g * 32 + _j
        _PERM[_g * 32 + 2 * _j + 1] = _g * 32 + 16 + _j


def _pair_body(a_ref, b_ref, out_ref):
    s = a_ref[0][:, None, :] + b_ref[0][None, :, :]
    out_ref[0] = s.astype(jnp.bfloat16)


@jax.jit
def _build_pairs(tables_perm):
    out = pl.pallas_call(
        _pair_body,
        grid=(NPAIR,),
        in_specs=[
            pl.BlockSpec((1, V, D), lambda k: (2 * k, 0, 0)),
            pl.BlockSpec((1, V, D), lambda k: (2 * k + 1, 0, 0)),
        ],
        out_specs=pl.BlockSpec((1, V, V, D), lambda k: (k, 0, 0, 0)),
        out_shape=jax.ShapeDtypeStruct((NPAIR, V, V, D), jnp.bfloat16),
    )(tables_perm, tables_perm)
    return out.reshape(NPAIR * VP, D)


def _body(idx_hbm, ptab_hbm, stab_hbm, out_hbm, idx_v, pidx_v, buf, out_v,
          *sems):
    wid = lax.axis_index("s") * NC + lax.axis_index("c")
    base0 = wid * NPW
    sg = sems[:NBUF]
    ss = sems[NBUF:]

    @pl.when(wid < NW - 1)
    def _():
        for f in range(F):
            pltpu.sync_copy(idx_hbm.at[f, pl.ds(base0, NPW)], idx_v.at[f])

    @pl.when(wid == NW - 1)
    def _():
        for f in range(F):
            pltpu.sync_copy(idx_hbm.at[f, pl.ds(base0, TAIL)],
                            idx_v.at[f, pl.ds(0, TAIL)])

    def mk_pidx(c, carry):
        sl = pl.ds(c * 16, 16)
        for k in range(NPAIR):
            pidx_v[k, sl] = (idx_v[2 * k, sl] * V + idx_v[2 * k + 1, sl]
                             + k * VP)
        return carry
    lax.fori_loop(0, TAIL // 16, mk_pidx, 0)

    @pl.when(wid < NW - 1)
    def _():
        def mk_pidx_tail(c, carry):
            sl = pl.ds(c * 16, 16)
            for k in range(NPAIR):
                pidx_v[k, sl] = (idx_v[2 * k, sl] * V + idx_v[2 * k + 1, sl]
                                 + k * VP)
            return carry
        lax.fori_loop(TAIL // 16, NPW // 16, mk_pidx_tail, 0)

    def valid(j):
        return jnp.logical_and(j < BLK, base0 + j * NB < N)

    def gathers(j, b):
        descs = [
            pltpu.make_async_copy(ptab_hbm.at[pidx_v.at[k, pl.ds(j * NB, NB)]],
                                  buf.at[b, k], sg[b])
            for k in range(NPAIR)
        ]
        descs.append(
            pltpu.make_async_copy(stab_hbm.at[idx_v.at[F - 1,
                                                       pl.ds(j * NB, NB)]],
                                  buf.at[b, NPAIR], sg[b]))
        return descs

    def fire_gathers(j, b):
        for k in range(NPAIR):
            pltpu.async_copy(ptab_hbm.at[pidx_v.at[k, pl.ds(j * NB, NB)]],
                             buf.at[b, k], sg[b])
        pltpu.async_copy(stab_hbm.at[idx_v.at[F - 1, pl.ds(j * NB, NB)]],
                         buf.at[b, NPAIR], sg[b])

    def store_desc(j, b):
        return pltpu.make_async_copy(out_v.at[b],
                                     out_hbm.at[pl.ds(base0 + j * NB, NB)],
                                     ss[b])

    for j0 in range(NBUF - 1):
        fire_gathers(j0, j0)

    def group(jg, c):
        for b in range(NBUF):
            j = jg * NBUF + b

            @pl.when(jnp.logical_and(j >= NBUF, valid(j - NBUF)))
            def _():
                store_desc(j - NBUF, b).wait()

            @pl.when(valid(j + NBUF - 1))
            def _():
                fire_gathers(j + NBUF - 1, (b + NBUF - 1) % NBUF)

            @pl.when(valid(j))
            def _():
                for dsc in gathers(j, b):
                    dsc.wait()

                def acc_row(r, cc):
                    for ch in range(D // 32):
                        sl = pl.ds(ch * 32, 32)
                        a, bb = plsc.unpack(
                            buf[b, 0, r, sl],
                            format=plsc.PackFormat.INTERLEAVED)
                        for k in range(1, NG):
                            ak, bk = plsc.unpack(
                                buf[b, k, r, sl],
                                format=plsc.PackFormat.INTERLEAVED)
                            a = a + ak
                            bb = bb + bk
                        out_v[b, r, pl.ds(ch * 32, 16)] = a
                        out_v[b, r, pl.ds(ch * 32 + 16, 16)] = bb
                    return cc
                lax.fori_loop(0, NB, acc_row, 0)

                pltpu.async_copy(out_v.at[b],
                                 out_hbm.at[pl.ds(base0 + j * NB, NB)], ss[b])
        return c

    lax.fori_loop(0, BLK // NBUF + 2, group, 0)


@jax.jit
def _sc_embed(node_features, pair_tables, single_table):
    return pl.kernel(
        _body,
        out_type=jax.ShapeDtypeStruct((N, D), jnp.float32),
        mesh=plsc.VectorSubcoreMesh(core_axis_name="c", subcore_axis_name="s"),
        scratch_types=[
            pltpu.VMEM((F, NPW), jnp.int32),
            pltpu.VMEM((NPAIR, NPW), jnp.int32),
            pltpu.VMEM((NBUF, NG, NB, D), jnp.bfloat16),
            pltpu.VMEM((NBUF, NB, D), jnp.float32),
        ] + [pltpu.SemaphoreType.DMA] * (2 * NBUF),
        compiler_params=pltpu.CompilerParams(use_tc_tiling_on_sc=False,
                                             needs_layout_passes=False),
    )(node_features, pair_tables, single_table)


def kernel(node_features, tables):
    tables_perm = tables[:, :, _PERM]
    pair_tables = _build_pairs(tables_perm)
    single_table = tables_perm[F - 1].astype(jnp.bfloat16)
    return _sc_embed(node_features, pair_tables, single_table)


# speedup vs baseline: 1.0963x; 1.0963x over previous
import numpy as np
import jax
import jax.numpy as jnp
from jax import lax
from jax.experimental import pallas as pl
from jax.experimental.pallas import tpu as pltpu, tpu_sc as plsc

F = 9
V = 124
D = 128
N = 50000
NPAIR = 4
VP = V * V
NC, NS = 2, 16
NW = NC * NS
NB = 16
BLK = 98
NPW = NB * BLK
TAIL = N - (NW - 1) * NPW
NBUF = 4
NG = NPAIR + 1
IPT = 16

_PERM = np.empty((D,), dtype=np.int32)
for _g in range(D // 32):
    for _j in range(16):
        _PERM[_g * 32 + 2 * _j] = _g * 32 + _j
        _PERM[_g * 32 + 2 * _j + 1] = _g * 32 + 16 + _j


def _build_body(tab_hbm, p_hbm, ta_v, tb_v, row_v, s0, s1):
    wid = lax.axis_index("s") * NC + lax.axis_index("c")
    k = wid // 8
    s = wid % 8
    i0 = s * IPT
    ni = jnp.where(s == 7, V - 7 * IPT, IPT)
    sem = (s0, s1)

    pltpu.sync_copy(tab_hbm.at[2 * k + 1], tb_v)

    @pl.when(s < 7)
    def _():
        pltpu.sync_copy(tab_hbm.at[2 * k, pl.ds(i0, IPT)], ta_v)

    @pl.when(s == 7)
    def _():
        pltpu.sync_copy(tab_hbm.at[2 * k, pl.ds(i0, V - 7 * IPT)],
                        ta_v.at[pl.ds(0, V - 7 * IPT)])

    def do_pair(ip, carry):
        for b in range(2):
            ii = ip * 2 + b

            @pl.when(jnp.logical_and(ii >= 2, ii < ni))
            def _():
                pltpu.make_async_copy(row_v.at[b], p_hbm.at[pl.ds(0, V)],
                                      sem[b]).wait()

            @pl.when(ii < ni)
            def _():
                a = [ta_v[ii, pl.ds(ch * 16, 16)] for ch in range(D // 16)]

                def do_j(j, c):
                    for ch in range(D // 32):
                        lo = a[2 * ch] + tb_v[j, pl.ds(ch * 32, 16)]
                        hi = a[2 * ch + 1] + tb_v[j, pl.ds(ch * 32 + 16, 16)]
                        row_v[b, j, pl.ds(ch * 32, 32)] = plsc.pack(
                            lo, hi, format=plsc.PackFormat.INTERLEAVED)
                    return c
                lax.fori_loop(0, V, do_j, 0)

                pltpu.async_copy(
                    row_v.at[b],
                    p_hbm.at[pl.ds(k * VP + (i0 + ii) * V, V)], sem[b])
        return carry

    lax.fori_loop(0, IPT // 2, do_pair, 0)

    for b in range(2):
        pltpu.make_async_copy(row_v.at[b], p_hbm.at[pl.ds(0, V)],
                              sem[b]).wait()


@jax.jit
def _build(tables):
    return pl.kernel(
        _build_body,
        out_type=jax.ShapeDtypeStruct((NPAIR * VP, D), jnp.bfloat16),
        mesh=plsc.VectorSubcoreMesh(core_axis_name="c", subcore_axis_name="s"),
        scratch_types=[
            pltpu.VMEM((IPT, D), jnp.float32),
            pltpu.VMEM((V, D), jnp.float32),
            pltpu.VMEM((2, V, D), jnp.bfloat16),
            pltpu.SemaphoreType.DMA,
            pltpu.SemaphoreType.DMA,
        ],
        compiler_params=pltpu.CompilerParams(use_tc_tiling_on_sc=False,
                                             needs_layout_passes=False),
    )(tables)


def _body(idx_hbm, ptab_hbm, stab_hbm, out_hbm, idx_v, pidx_v, buf, out_v,
          *sems):
    wid = lax.axis_index("s") * NC + lax.axis_index("c")
    base0 = wid * NPW
    sg = sems[:NBUF]
    ss = sems[NBUF:]

    @pl.when(wid < NW - 1)
    def _():
        for f in range(F):
            pltpu.sync_copy(idx_hbm.at[f, pl.ds(base0, NPW)], idx_v.at[f])

    @pl.when(wid == NW - 1)
    def _():
        for f in range(F):
            pltpu.sync_copy(idx_hbm.at[f, pl.ds(base0, TAIL)],
                            idx_v.at[f, pl.ds(0, TAIL)])

    def mk_pidx(c, carry):
        sl = pl.ds(c * 16, 16)
        for k in range(NPAIR):
            pidx_v[k, sl] = (idx_v[2 * k, sl] * V + idx_v[2 * k + 1, sl]
                             + k * VP)
        return carry
    lax.fori_loop(0, TAIL // 16, mk_pidx, 0)

    @pl.when(wid < NW - 1)
    def _():
        def mk_pidx_tail(c, carry):
            sl = pl.ds(c * 16, 16)
            for k in range(NPAIR):
                pidx_v[k, sl] = (idx_v[2 * k, sl] * V + idx_v[2 * k + 1, sl]
                                 + k * VP)
            return carry
        lax.fori_loop(TAIL // 16, NPW // 16, mk_pidx_tail, 0)

    def valid(j):
        return jnp.logical_and(j < BLK, base0 + j * NB < N)

    def gathers(j, b):
        descs = [
            pltpu.make_async_copy(ptab_hbm.at[pidx_v.at[k, pl.ds(j * NB, NB)]],
                                  buf.at[b, k], sg[b])
            for k in range(NPAIR)
        ]
        descs.append(
            pltpu.make_async_copy(stab_hbm.at[idx_v.at[F - 1,
                                                       pl.ds(j * NB, NB)]],
                                  buf.at[b, NPAIR], sg[b]))
        return descs

    def fire_gathers(j, b):
        for k in range(NPAIR):
            pltpu.async_copy(ptab_hbm.at[pidx_v.at[k, pl.ds(j * NB, NB)]],
                             buf.at[b, k], sg[b])
        pltpu.async_copy(stab_hbm.at[idx_v.at[F - 1, pl.ds(j * NB, NB)]],
                         buf.at[b, NPAIR], sg[b])

    def store_desc(j, b):
        return pltpu.make_async_copy(out_v.at[b],
                                     out_hbm.at[pl.ds(base0 + j * NB, NB)],
                                     ss[b])

    for j0 in range(NBUF - 1):
        fire_gathers(j0, j0)

    def group(jg, c):
        for b in range(NBUF):
            j = jg * NBUF + b

            @pl.when(jnp.logical_and(j >= NBUF, valid(j - NBUF)))
            def _():
                store_desc(j - NBUF, b).wait()

            @pl.when(valid(j + NBUF - 1))
            def _():
                fire_gathers(j + NBUF - 1, (b + NBUF - 1) % NBUF)

            @pl.when(valid(j))
            def _():
                for dsc in gathers(j, b):
                    dsc.wait()

                def acc_row(r, cc):
                    for ch in range(D // 32):
                        sl = pl.ds(ch * 32, 32)
                        a, bb = plsc.unpack(
                            buf[b, 0, r, sl],
                            format=plsc.PackFormat.INTERLEAVED)
                        for k in range(1, NG):
                            ak, bk = plsc.unpack(
                                buf[b, k, r, sl],
                                format=plsc.PackFormat.INTERLEAVED)
                            a = a + ak
                            bb = bb + bk
                        out_v[b, r, pl.ds(ch * 32, 16)] = a
                        out_v[b, r, pl.ds(ch * 32 + 16, 16)] = bb
                    return cc
                lax.fori_loop(0, NB, acc_row, 0)

                pltpu.async_copy(out_v.at[b],
                                 out_hbm.at[pl.ds(base0 + j * NB, NB)], ss[b])
        return c

    lax.fori_loop(0, BLK // NBUF + 2, group, 0)


@jax.jit
def _sc_embed(node_features, pair_tables, single_table):
    return pl.kernel(
        _body,
        out_type=jax.ShapeDtypeStruct((N, D), jnp.float32),
        mesh=plsc.VectorSubcoreMesh(core_axis_name="c", subcore_axis_name="s"),
        scratch_types=[
            pltpu.VMEM((F, NPW), jnp.int32),
            pltpu.VMEM((NPAIR, NPW), jnp.int32),
            pltpu.VMEM((NBUF, NG, NB, D), jnp.bfloat16),
            pltpu.VMEM((NBUF, NB, D), jnp.float32),
        ] + [pltpu.SemaphoreType.DMA] * (2 * NBUF),
        compiler_params=pltpu.CompilerParams(use_tc_tiling_on_sc=False,
                                             needs_layout_passes=False),
    )(node_features, pair_tables, single_table)


def kernel(node_features, tables):
    pair_tables = _build(tables)
    single_table = tables[F - 1][:, _PERM].astype(jnp.bfloat16)
    return _sc_embed(node_features, pair_tables, single_table)


# speedup vs baseline: 1.1153x; 1.0173x over previous
import numpy as np
import jax
import jax.numpy as jnp
from jax import lax
from jax.experimental import pallas as pl
from jax.experimental.pallas import tpu as pltpu, tpu_sc as plsc

F = 9
V = 124
D = 128
N = 50000
NPAIR = 4
VP = V * V
NC, NS = 2, 16
NW = NC * NS
NB = 16
BLK = 98
NPW = NB * BLK
TAIL = N - (NW - 1) * NPW
NBUF = 4
NG = NPAIR + 1
IPT = 16

_PERM = np.empty((D,), dtype=np.int32)
for _g in range(D // 32):
    for _j in range(16):
        _PERM[_g * 32 + 2 * _j] = _g * 32 + _j
        _PERM[_g * 32 + 2 * _j + 1] = _g * 32 + 16 + _j


def _build_body(tab_hbm, p_hbm, ta_v, tb_v, row_v, s0, s1):
    wid = lax.axis_index("s") * NC + lax.axis_index("c")
    k = wid // 8
    s = wid % 8
    i0 = s * IPT
    ni = jnp.where(s == 7, V - 7 * IPT, IPT)
    sem = (s0, s1)

    pltpu.sync_copy(tab_hbm.at[2 * k + 1], tb_v)

    @pl.when(s < 7)
    def _():
        pltpu.sync_copy(tab_hbm.at[2 * k, pl.ds(i0, IPT)], ta_v)

    @pl.when(s == 7)
    def _():
        pltpu.sync_copy(tab_hbm.at[2 * k, pl.ds(i0, V - 7 * IPT)],
                        ta_v.at[pl.ds(0, V - 7 * IPT)])

    def do_pair(ip, carry):
        for b in range(2):
            ii = ip * 2 + b

            @pl.when(jnp.logical_and(ii >= 2, ii < ni))
            def _():
                pltpu.make_async_copy(row_v.at[b], p_hbm.at[pl.ds(0, V)],
                                      sem[b]).wait()

            @pl.when(ii < ni)
            def _():
                a = [ta_v[ii, pl.ds(ch * 16, 16)] for ch in range(D // 16)]

                def do_j(jq, c):
                    for u in range(4):
                        j = jq * 4 + u
                        for ch in range(D // 32):
                            lo = a[2 * ch] + tb_v[j, pl.ds(ch * 32, 16)]
                            hi = a[2 * ch + 1] + tb_v[j,
                                                      pl.ds(ch * 32 + 16, 16)]
                            row_v[b, j, pl.ds(ch * 32, 32)] = plsc.pack(
                                lo, hi, format=plsc.PackFormat.INTERLEAVED)
                    return c
                lax.fori_loop(0, V // 4, do_j, 0)

                pltpu.async_copy(
                    row_v.at[b],
                    p_hbm.at[pl.ds(k * VP + (i0 + ii) * V, V)], sem[b])
        return carry

    lax.fori_loop(0, IPT // 2, do_pair, 0)

    for b in range(2):
        pltpu.make_async_copy(row_v.at[b], p_hbm.at[pl.ds(0, V)],
                              sem[b]).wait()


@jax.jit
def _build(tables):
    return pl.kernel(
        _build_body,
        out_type=jax.ShapeDtypeStruct((NPAIR * VP, D), jnp.bfloat16),
        mesh=plsc.VectorSubcoreMesh(core_axis_name="c", subcore_axis_name="s"),
        scratch_types=[
            pltpu.VMEM((IPT, D), jnp.float32),
            pltpu.VMEM((V, D), jnp.float32),
            pltpu.VMEM((2, V, D), jnp.bfloat16),
            pltpu.SemaphoreType.DMA,
            pltpu.SemaphoreType.DMA,
        ],
        compiler_params=pltpu.CompilerParams(use_tc_tiling_on_sc=False,
                                             needs_layout_passes=False),
    )(tables)


def _body(idx_hbm, ptab_hbm, stab_hbm, out_hbm, idx_v, pidx_v, buf, out_v,
          *sems):
    wid = lax.axis_index("s") * NC + lax.axis_index("c")
    base0 = wid * NPW
    sg = sems[:NBUF]
    ss = sems[NBUF:]

    @pl.when(wid < NW - 1)
    def _():
        for f in range(F):
            pltpu.sync_copy(idx_hbm.at[f, pl.ds(base0, NPW)], idx_v.at[f])

    @pl.when(wid == NW - 1)
    def _():
        for f in range(F):
            pltpu.sync_copy(idx_hbm.at[f, pl.ds(base0, TAIL)],
                            idx_v.at[f, pl.ds(0, TAIL)])

    def mk_pidx(c, carry):
        sl = pl.ds(c * 16, 16)
        for k in range(NPAIR):
            pidx_v[k, sl] = (idx_v[2 * k, sl] * V + idx_v[2 * k + 1, sl]
                             + k * VP)
        return carry
    lax.fori_loop(0, TAIL // 16, mk_pidx, 0)

    @pl.when(wid < NW - 1)
    def _():
        def mk_pidx_tail(c, carry):
            sl = pl.ds(c * 16, 16)
            for k in range(NPAIR):
                pidx_v[k, sl] = (idx_v[2 * k, sl] * V + idx_v[2 * k + 1, sl]
                                 + k * VP)
            return carry
        lax.fori_loop(TAIL // 16, NPW // 16, mk_pidx_tail, 0)

    def valid(j):
        return jnp.logical_and(j < BLK, base0 + j * NB < N)

    def gathers(j, b):
        descs = [
            pltpu.make_async_copy(ptab_hbm.at[pidx_v.at[k, pl.ds(j * NB, NB)]],
                                  buf.at[b, k], sg[b])
            for k in range(NPAIR)
        ]
        descs.append(
            pltpu.make_async_copy(stab_hbm.at[idx_v.at[F - 1,
                                                       pl.ds(j * NB, NB)]],
                                  buf.at[b, NPAIR], sg[b]))
        return descs

    def fire_gathers(j, b):
        for k in range(NPAIR):
            pltpu.async_copy(ptab_hbm.at[pidx_v.at[k, pl.ds(j * NB, NB)]],
                             buf.at[b, k], sg[b])
        pltpu.async_copy(stab_hbm.at[idx_v.at[F - 1, pl.ds(j * NB, NB)]],
                         buf.at[b, NPAIR], sg[b])

    def store_desc(j, b):
        return pltpu.make_async_copy(out_v.at[b],
                                     out_hbm.at[pl.ds(base0 + j * NB, NB)],
                                     ss[b])

    for j0 in range(NBUF - 1):
        fire_gathers(j0, j0)

    def group(jg, c):
        for b in range(NBUF):
            j = jg * NBUF + b

            @pl.when(jnp.logical_and(j >= NBUF, valid(j - NBUF)))
            def _():
                store_desc(j - NBUF, b).wait()

            @pl.when(valid(j + NBUF - 1))
            def _():
                fire_gathers(j + NBUF - 1, (b + NBUF - 1) % NBUF)

            @pl.when(valid(j))
            def _():
                for dsc in gathers(j, b):
                    dsc.wait()

                def acc_row(r, cc):
                    for ch in range(D // 32):
                        sl = pl.ds(ch * 32, 32)
                        a, bb = plsc.unpack(
                            buf[b, 0, r, sl],
                            format=plsc.PackFormat.INTERLEAVED)
                        for k in range(1, NG):
                            ak, bk = plsc.unpack(
                                buf[b, k, r, sl],
                                format=plsc.PackFormat.INTERLEAVED)
                            a = a + ak
                            bb = bb + bk
                        out_v[b, r, pl.ds(ch * 32, 16)] = a
                        out_v[b, r, pl.ds(ch * 32 + 16, 16)] = bb
                    return cc
                lax.fori_loop(0, NB, acc_row, 0)

                pltpu.async_copy(out_v.at[b],
                                 out_hbm.at[pl.ds(base0 + j * NB, NB)], ss[b])
        return c

    lax.fori_loop(0, BLK // NBUF + 2, group, 0)


@jax.jit
def _sc_embed(node_features, pair_tables, single_table):
    return pl.kernel(
        _body,
        out_type=jax.ShapeDtypeStruct((N, D), jnp.float32),
        mesh=plsc.VectorSubcoreMesh(core_axis_name="c", subcore_axis_name="s"),
        scratch_types=[
            pltpu.VMEM((F, NPW), jnp.int32),
            pltpu.VMEM((NPAIR, NPW), jnp.int32),
            pltpu.VMEM((NBUF, NG, NB, D), jnp.bfloat16),
            pltpu.VMEM((NBUF, NB, D), jnp.float32),
        ] + [pltpu.SemaphoreType.DMA] * (2 * NBUF),
        compiler_params=pltpu.CompilerParams(use_tc_tiling_on_sc=False,
                                             needs_layout_passes=False),
    )(node_features, pair_tables, single_table)


def kernel(node_features, tables):
    pair_tables = _build(tables)
    single_table = tables[F - 1][:, _PERM].astype(jnp.bfloat16)
    return _sc_embed(node_features, pair_tables, single_table)


# speedup vs baseline: 1.5253x; 1.3676x over previous
import numpy as np
import jax
import jax.numpy as jnp
from jax import lax
from jax.experimental import pallas as pl
from jax.experimental.pallas import tpu as pltpu, tpu_sc as plsc

F = 9
V = 124
D = 128
N = 50000
NPAIR = 4
VP = V * V
NC, NS = 2, 16
NW = NC * NS
NB = 16
BLK = 98
NPW = NB * BLK
TAIL = N - (NW - 1) * NPW
NBUF = 4
NG = NPAIR + 1
IPT = 16

_PERM = np.empty((D,), dtype=np.int32)
for _g in range(D // 32):
    for _j in range(16):
        _PERM[_g * 32 + 2 * _j] = _g * 32 + _j
        _PERM[_g * 32 + 2 * _j + 1] = _g * 32 + 16 + _j


def _build_body(tab_hbm, p_hbm, ta_v, tb_v, row_v, s0, s1):
    wid = lax.axis_index("s") * NC + lax.axis_index("c")
    k = wid // 8
    s = wid % 8
    i0 = s * IPT
    ni = jnp.where(s == 7, V - 7 * IPT, IPT)
    sem = (s0, s1)

    pltpu.sync_copy(tab_hbm.at[2 * k + 1], tb_v)

    @pl.when(s < 7)
    def _():
        pltpu.sync_copy(tab_hbm.at[2 * k, pl.ds(i0, IPT)], ta_v)

    @pl.when(s == 7)
    def _():
        pltpu.sync_copy(tab_hbm.at[2 * k, pl.ds(i0, V - 7 * IPT)],
                        ta_v.at[pl.ds(0, V - 7 * IPT)])

    def do_pair(ip, carry):
        for b in range(2):
            ii = ip * 2 + b

            @pl.when(jnp.logical_and(ii >= 2, ii < ni))
            def _():
                pltpu.make_async_copy(row_v.at[b], p_hbm.at[pl.ds(0, V)],
                                      sem[b]).wait()

            @pl.when(ii < ni)
            def _():
                a = [ta_v[ii, pl.ds(ch * 32, 32)] for ch in range(D // 32)]

                def do_j(jq, c):
                    for u in range(4):
                        j = jq * 4 + u
                        for ch in range(D // 32):
                            sl = pl.ds(ch * 32, 32)
                            row_v[b, j, sl] = a[ch] + tb_v[j, sl]
                    return c
                lax.fori_loop(0, V // 4, do_j, 0)

                pltpu.async_copy(
                    row_v.at[b],
                    p_hbm.at[pl.ds(k * VP + (i0 + ii) * V, V)], sem[b])
        return carry

    lax.fori_loop(0, IPT // 2, do_pair, 0)

    for b in range(2):
        pltpu.make_async_copy(row_v.at[b], p_hbm.at[pl.ds(0, V)],
                              sem[b]).wait()


@jax.jit
def _build(tables):
    return pl.kernel(
        _build_body,
        out_type=jax.ShapeDtypeStruct((NPAIR * VP, D), jnp.bfloat16),
        mesh=plsc.VectorSubcoreMesh(core_axis_name="c", subcore_axis_name="s"),
        scratch_types=[
            pltpu.VMEM((IPT, D), jnp.bfloat16),
            pltpu.VMEM((V, D), jnp.bfloat16),
            pltpu.VMEM((2, V, D), jnp.bfloat16),
            pltpu.SemaphoreType.DMA,
            pltpu.SemaphoreType.DMA,
        ],
        compiler_params=pltpu.CompilerParams(use_tc_tiling_on_sc=False,
                                             needs_layout_passes=False),
    )(tables)


def _body(idx_hbm, ptab_hbm, stab_hbm, out_hbm, idx_v, pidx_v, buf, out_v,
          *sems):
    wid = lax.axis_index("s") * NC + lax.axis_index("c")
    base0 = wid * NPW
    sg = sems[:NBUF]
    ss = sems[NBUF:]

    @pl.when(wid < NW - 1)
    def _():
        for f in range(F):
            pltpu.sync_copy(idx_hbm.at[f, pl.ds(base0, NPW)], idx_v.at[f])

    @pl.when(wid == NW - 1)
    def _():
        for f in range(F):
            pltpu.sync_copy(idx_hbm.at[f, pl.ds(base0, TAIL)],
                            idx_v.at[f, pl.ds(0, TAIL)])

    def mk_pidx(c, carry):
        sl = pl.ds(c * 16, 16)
        for k in range(NPAIR):
            pidx_v[k, sl] = (idx_v[2 * k, sl] * V + idx_v[2 * k + 1, sl]
                             + k * VP)
        return carry
    lax.fori_loop(0, TAIL // 16, mk_pidx, 0)

    @pl.when(wid < NW - 1)
    def _():
        def mk_pidx_tail(c, carry):
            sl = pl.ds(c * 16, 16)
            for k in range(NPAIR):
                pidx_v[k, sl] = (idx_v[2 * k, sl] * V + idx_v[2 * k + 1, sl]
                                 + k * VP)
            return carry
        lax.fori_loop(TAIL // 16, NPW // 16, mk_pidx_tail, 0)

    def valid(j):
        return jnp.logical_and(j < BLK, base0 + j * NB < N)

    def gathers(j, b):
        descs = [
            pltpu.make_async_copy(ptab_hbm.at[pidx_v.at[k, pl.ds(j * NB, NB)]],
                                  buf.at[b, k], sg[b])
            for k in range(NPAIR)
        ]
        descs.append(
            pltpu.make_async_copy(stab_hbm.at[idx_v.at[F - 1,
                                                       pl.ds(j * NB, NB)]],
                                  buf.at[b, NPAIR], sg[b]))
        return descs

    def fire_gathers(j, b):
        for k in range(NPAIR):
            pltpu.async_copy(ptab_hbm.at[pidx_v.at[k, pl.ds(j * NB, NB)]],
                             buf.at[b, k], sg[b])
        pltpu.async_copy(stab_hbm.at[idx_v.at[F - 1, pl.ds(j * NB, NB)]],
                         buf.at[b, NPAIR], sg[b])

    def store_desc(j, b):
        return pltpu.make_async_copy(out_v.at[b],
                                     out_hbm.at[pl.ds(base0 + j * NB, NB)],
                                     ss[b])

    for j0 in range(NBUF - 1):
        fire_gathers(j0, j0)

    def group(jg, c):
        for b in range(NBUF):
            j = jg * NBUF + b

            @pl.when(jnp.logical_and(j >= NBUF, valid(j - NBUF)))
            def _():
                store_desc(j - NBUF, b).wait()

            @pl.when(valid(j + NBUF - 1))
            def _():
                fire_gathers(j + NBUF - 1, (b + NBUF - 1) % NBUF)

            @pl.when(valid(j))
            def _():
                for dsc in gathers(j, b):
                    dsc.wait()

                def acc_row(r, cc):
                    for ch in range(D // 32):
                        sl = pl.ds(ch * 32, 32)
                        a, bb = plsc.unpack(
                            buf[b, 0, r, sl],
                            format=plsc.PackFormat.INTERLEAVED)
                        for k in range(1, NG):
                            ak, bk = plsc.unpack(
                                buf[b, k, r, sl],
                                format=plsc.PackFormat.INTERLEAVED)
                            a = a + ak
                            bb = bb + bk
                        out_v[b, r, pl.ds(ch * 32, 16)] = a
                        out_v[b, r, pl.ds(ch * 32 + 16, 16)] = bb
                    return cc
                lax.fori_loop(0, NB, acc_row, 0)

                pltpu.async_copy(out_v.at[b],
                                 out_hbm.at[pl.ds(base0 + j * NB, NB)], ss[b])
        return c

    lax.fori_loop(0, BLK // NBUF + 2, group, 0)


@jax.jit
def _sc_embed(node_features, pair_tables, single_table):
    return pl.kernel(
        _body,
        out_type=jax.ShapeDtypeStruct((N, D), jnp.float32),
        mesh=plsc.VectorSubcoreMesh(core_axis_name="c", subcore_axis_name="s"),
        scratch_types=[
            pltpu.VMEM((F, NPW), jnp.int32),
            pltpu.VMEM((NPAIR, NPW), jnp.int32),
            pltpu.VMEM((NBUF, NG, NB, D), jnp.bfloat16),
            pltpu.VMEM((NBUF, NB, D), jnp.float32),
        ] + [pltpu.SemaphoreType.DMA] * (2 * NBUF),
        compiler_params=pltpu.CompilerParams(use_tc_tiling_on_sc=False,
                                             needs_layout_passes=False),
    )(node_features, pair_tables, single_table)


def kernel(node_features, tables):
    tables_pb = tables[:, :, _PERM].astype(jnp.bfloat16)
    pair_tables = _build(tables_pb)
    single_table = tables_pb[F - 1]
    return _sc_embed(node_features, pair_tables, single_table)
